# Initial kernel scaffold; baseline (speedup 1.0000x reference)
import jax
import jax.numpy as jnp
from jax.experimental import pallas as pl


def kernel(x1, x2, x3, t, mask, device, emb1, emb2, emb3, W1, b1, W2, b2, W3, b3):
    raise NotImplementedError("write your pallas kernel here")


# speedup vs baseline: 1.4650x; 1.4650x over previous
import functools

import jax
import jax.numpy as jnp
from jax import lax
from jax.experimental import pallas as pl
from jax.experimental.pallas import tpu as pltpu
from jax.experimental.pallas import tpu_sc as plsc

_B = 4096
_L = 50
_TT = 257
_MD = 256
_SLOT = 264
_NROWS = _L * _SLOT
_NIDX = 160
_ZROW = 257


def _pre_body(bd_ref, w_ref, out_ref):
    out_ref[0] = jnp.dot(bd_ref[...], w_ref[0],
                         preferred_element_type=jnp.float32)


def _idx_body(x1_ref, x2_ref, x3_ref, out_ref):
    l = lax.broadcasted_iota(jnp.int32, x1_ref.shape, 1)
    base = l * _SLOT
    pad = jnp.full((x1_ref.shape[0], _NIDX - 3 * _L), _ZROW, jnp.int32)
    out_ref[...] = jnp.concatenate(
        [x1_ref[...] + base,
         x2_ref[...] + base + 104,
         x3_ref[...] + base + 208,
         pad], axis=1)


def _mlp_body(acc_ref, t_ref, wt_ref, b1_ref, w2_ref, b2_ref, w3_ref,
              b3_ref, out_ref):
    h = (acc_ref[...]
         + jnp.dot(t_ref[...], wt_ref[...],
                   preferred_element_type=jnp.float32)
         + b1_ref[...])
    h = jnp.maximum(h, 0.0)
    h = jnp.maximum(
        jnp.dot(h, w2_ref[...], preferred_element_type=jnp.float32)
        + b2_ref[...], 0.0)
    out_ref[...] = (jnp.dot(h, w3_ref[...],
                            preferred_element_type=jnp.float32)
                    + b3_ref[...])


def _gather_body(table_hbm, idx_hbm, out_hbm, idx_v, buf_v, out_v, sem):
    wid = lax.axis_index("s") * 2 + lax.axis_index("c")
    for sub in range(2):
        b0 = wid * 128 + sub * 64
        i0 = pl.multiple_of(b0 * _NIDX, 8)
        pltpu.sync_copy(idx_hbm.at[pl.ds(i0, 64 * _NIDX)], idx_v)

        def bbody(b, carry):
            o1 = pl.multiple_of(b * _NIDX, 8)
            o2 = pl.multiple_of(b * _NIDX + 80, 8)
            cp1 = pltpu.async_copy(table_hbm.at[idx_v.at[pl.ds(o1, 80)]],
                                   buf_v.at[pl.ds(0, 80)], sem)
            cp2 = pltpu.async_copy(table_hbm.at[idx_v.at[pl.ds(o2, 80)]],
                                   buf_v.at[pl.ds(80, 80)], sem)
            cp1.wait()
            cp2.wait()

            def rbody(r, accs):
                return tuple(accs[j] + buf_v[r, pl.ds(16 * j, 16)]
                             for j in range(16))

            accs = lax.fori_loop(
                0, _NIDX, rbody,
                tuple(jnp.zeros((16,), jnp.float32) for _ in range(16)))
            for j in range(16):
                ob = pl.multiple_of(b * _MD + 16 * j, 8)
                out_v[pl.ds(ob, 16)] = accs[j]
            return carry

        lax.fori_loop(0, 64, bbody, 0)
        oo = pl.multiple_of(b0 * _MD, 8)
        pltpu.sync_copy(out_v, out_hbm.at[pl.ds(oo, 64 * _MD)])


def _make_gather_sum():
    mesh = plsc.VectorSubcoreMesh(core_axis_name="c", subcore_axis_name="s")
    return pl.kernel(
        _gather_body,
        out_type=jax.ShapeDtypeStruct((_B * _MD,), jnp.float32),
        mesh=mesh,
        scratch_types=[
            pltpu.VMEM((64 * _NIDX,), jnp.int32),
            pltpu.VMEM((_NIDX, _MD), jnp.float32),
            pltpu.VMEM((64 * _MD,), jnp.float32),
            pltpu.SemaphoreType.DMA,
        ],
    )


def kernel(x1, x2, x3, t, mask, device, emb1, emb2, emb3, W1, b1, W2, b2,
           W3, b3):
    del mask, device
    x1 = x1.astype(jnp.int32)
    x2 = x2.astype(jnp.int32)
    x3 = x3.astype(jnp.int32)
    W1r = W1.reshape(_L, _TT, _MD)

    bd = jnp.zeros((_SLOT, _TT), jnp.float32)
    bd = bd.at[0:101, 0:96].set(emb1)
    bd = bd.at[104:205, 96:192].set(emb2)
    bd = bd.at[208:257, 192:256].set(emb3)

    u = pl.pallas_call(
        _pre_body,
        grid=(_L,),
        in_specs=[
            pl.BlockSpec((_SLOT, _TT), lambda l: (0, 0)),
            pl.BlockSpec((1, _TT, _MD), lambda l: (l, 0, 0)),
        ],
        out_specs=pl.BlockSpec((1, _SLOT, _MD), lambda l: (l, 0, 0)),
        out_shape=jax.ShapeDtypeStruct((_L, _SLOT, _MD), jnp.float32),
    )(bd, W1r)
    table = u.reshape(_NROWS, _MD)

    idx = pl.pallas_call(
        _idx_body,
        grid=(_B // 512,),
        in_specs=[pl.BlockSpec((512, _L), lambda i: (i, 0))] * 3,
        out_specs=pl.BlockSpec((512, _NIDX), lambda i: (i, 0)),
        out_shape=jax.ShapeDtypeStruct((_B, _NIDX), jnp.int32),
    )(x1, x2, x3)

    acc = _make_gather_sum()(table, idx.reshape(_B * _NIDX)).reshape(_B, _MD)

    wt = W1r[:, 256, :]
    out = pl.pallas_call(
        _mlp_body,
        grid=(_B // 512,),
        in_specs=[
            pl.BlockSpec((512, _MD), lambda i: (i, 0)),
            pl.BlockSpec((512, _L), lambda i: (i, 0)),
            pl.BlockSpec((_L, _MD), lambda i: (0, 0)),
            pl.BlockSpec((1, _MD), lambda i: (0, 0)),
            pl.BlockSpec((_MD, _MD), lambda i: (0, 0)),
            pl.BlockSpec((1, _MD), lambda i: (0, 0)),
            pl.BlockSpec((_MD, 1), lambda i: (0, 0)),
            pl.BlockSpec((1, 1), lambda i: (0, 0)),
        ],
        out_specs=pl.BlockSpec((512, 1), lambda i: (i, 0)),
        out_shape=jax.ShapeDtypeStruct((_B, 1), jnp.float32),
    )(acc, t, wt, b1.reshape(1, _MD), W2, b2.reshape(1, _MD), W3,
      b3.reshape(1, 1))
    return out


# speedup vs baseline: 1.4659x; 1.0006x over previous
import functools

import jax
import jax.numpy as jnp
from jax import lax
from jax.experimental import pallas as pl
from jax.experimental.pallas import tpu as pltpu
from jax.experimental.pallas import tpu_sc as plsc

_B = 4096
_L = 50
_TT = 257
_MD = 256
_SLOT = 264
_NROWS = _L * _SLOT
_NIDX = 160
_ZROW = 257


def _pre_body(bd_ref, w_ref, out_ref):
    out_ref[0] = jnp.dot(bd_ref[...], w_ref[0],
                         preferred_element_type=jnp.float32)


def _idx_body(x1_ref, x2_ref, x3_ref, out_ref):
    l = lax.broadcasted_iota(jnp.int32, x1_ref.shape, 1)
    base = l * _SLOT
    pad = jnp.full((x1_ref.shape[0], _NIDX - 3 * _L), _ZROW, jnp.int32)
    out_ref[...] = jnp.concatenate(
        [x1_ref[...] + base,
         x2_ref[...] + base + 104,
         x3_ref[...] + base + 208,
         pad], axis=1)


def _mlp_body(acc_ref, t_ref, wt_ref, b1_ref, w2_ref, b2_ref, w3_ref,
              b3_ref, out_ref):
    h = (acc_ref[...]
         + jnp.dot(t_ref[...], wt_ref[...],
                   preferred_element_type=jnp.float32)
         + b1_ref[...])
    h = jnp.maximum(h, 0.0)
    h = jnp.maximum(
        jnp.dot(h, w2_ref[...], preferred_element_type=jnp.float32)
        + b2_ref[...], 0.0)
    out_ref[...] = (jnp.dot(h, w3_ref[...],
                            preferred_element_type=jnp.float32)
                    + b3_ref[...])


def _gather_body(table_hbm, idx_hbm, out_hbm, idx_v, buf_v, out_v, sem0,
                 sem1):
    sems = (sem0, sem1)
    wid = lax.axis_index("s") * 2 + lax.axis_index("c")

    def fire(b, slot):
        o1 = pl.multiple_of(b * _NIDX, 8)
        o2 = pl.multiple_of(b * _NIDX + 80, 8)
        pltpu.async_copy(table_hbm.at[idx_v.at[pl.ds(o1, 80)]],
                         buf_v.at[slot, pl.ds(0, 80)], sems[slot])
        pltpu.async_copy(table_hbm.at[idx_v.at[pl.ds(o2, 80)]],
                         buf_v.at[slot, pl.ds(80, 80)], sems[slot])

    def wait_slot(slot):
        pltpu.make_async_copy(table_hbm.at[pl.ds(0, _NIDX)],
                              buf_v.at[slot], sems[slot]).wait()

    def reduce_store(b, slot):
        def rbody(r, accs):
            return tuple(accs[j] + buf_v[slot, r, pl.ds(16 * j, 16)]
                         for j in range(16))

        accs = lax.fori_loop(
            0, _NIDX, rbody,
            tuple(jnp.zeros((16,), jnp.float32) for _ in range(16)))
        for j in range(16):
            ob = pl.multiple_of(b * _MD + 16 * j, 8)
            out_v[pl.ds(ob, 16)] = accs[j]

    for sub in range(2):
        b0 = wid * 128 + sub * 64
        i0 = pl.multiple_of(b0 * _NIDX, 8)
        pltpu.sync_copy(idx_hbm.at[pl.ds(i0, 64 * _NIDX)], idx_v)
        fire(0, 0)
        fire(1, 1)

        def pair(bb, carry):
            b = bb * 2
            wait_slot(0)
            reduce_store(b, 0)

            @pl.when(bb < 31)
            def _():
                fire(b + 2, 0)

            wait_slot(1)
            reduce_store(b + 1, 1)

            @pl.when(bb < 31)
            def _():
                fire(b + 3, 1)

            return carry

        lax.fori_loop(0, 32, pair, 0)
        oo = pl.multiple_of(b0 * _MD, 8)
        pltpu.sync_copy(out_v, out_hbm.at[pl.ds(oo, 64 * _MD)])


def _make_gather_sum():
    mesh = plsc.VectorSubcoreMesh(core_axis_name="c", subcore_axis_name="s")
    return pl.kernel(
        _gather_body,
        out_type=jax.ShapeDtypeStruct((_B * _MD,), jnp.float32),
        mesh=mesh,
        scratch_types=[
            pltpu.VMEM((64 * _NIDX,), jnp.int32),
            pltpu.VMEM((2, _NIDX, _MD), jnp.float32),
            pltpu.VMEM((64 * _MD,), jnp.float32),
            pltpu.SemaphoreType.DMA,
            pltpu.SemaphoreType.DMA,
        ],
        compiler_params=pltpu.CompilerParams(use_tc_tiling_on_sc=False),
    )


def kernel(x1, x2, x3, t, mask, device, emb1, emb2, emb3, W1, b1, W2, b2,
           W3, b3):
    del mask, device
    x1 = x1.astype(jnp.int32)
    x2 = x2.astype(jnp.int32)
    x3 = x3.astype(jnp.int32)
    W1r = W1.reshape(_L, _TT, _MD)

    bd = jnp.zeros((_SLOT, _TT), jnp.float32)
    bd = bd.at[0:101, 0:96].set(emb1)
    bd = bd.at[104:205, 96:192].set(emb2)
    bd = bd.at[208:257, 192:256].set(emb3)

    u = pl.pallas_call(
        _pre_body,
        grid=(_L,),
        in_specs=[
            pl.BlockSpec((_SLOT, _TT), lambda l: (0, 0)),
            pl.BlockSpec((1, _TT, _MD), lambda l: (l, 0, 0)),
        ],
        out_specs=pl.BlockSpec((1, _SLOT, _MD), lambda l: (l, 0, 0)),
        out_shape=jax.ShapeDtypeStruct((_L, _SLOT, _MD), jnp.float32),
    )(bd, W1r)
    table = u.reshape(_NROWS, _MD)

    idx = pl.pallas_call(
        _idx_body,
        grid=(_B // 512,),
        in_specs=[pl.BlockSpec((512, _L), lambda i: (i, 0))] * 3,
        out_specs=pl.BlockSpec((512, _NIDX), lambda i: (i, 0)),
        out_shape=jax.ShapeDtypeStruct((_B, _NIDX), jnp.int32),
    )(x1, x2, x3)

    acc = _make_gather_sum()(table, idx.reshape(_B * _NIDX)).reshape(_B, _MD)

    wt = W1r[:, 256, :]
    out = pl.pallas_call(
        _mlp_body,
        grid=(_B // 512,),
        in_specs=[
            pl.BlockSpec((512, _MD), lambda i: (i, 0)),
            pl.BlockSpec((512, _L), lambda i: (i, 0)),
            pl.BlockSpec((_L, _MD), lambda i: (0, 0)),
            pl.BlockSpec((1, _MD), lambda i: (0, 0)),
            pl.BlockSpec((_MD, _MD), lambda i: (0, 0)),
            pl.BlockSpec((1, _MD), lambda i: (0, 0)),
            pl.BlockSpec((_MD, 1), lambda i: (0, 0)),
            pl.BlockSpec((1, 1), lambda i: (0, 0)),
        ],
        out_specs=pl.BlockSpec((512, 1), lambda i: (i, 0)),
        out_shape=jax.ShapeDtypeStruct((_B, 1), jnp.float32),
    )(acc, t, wt, b1.reshape(1, _MD), W2, b2.reshape(1, _MD), W3,
      b3.reshape(1, 1))
    return out
